# Initial kernel scaffold; baseline (speedup 1.0000x reference)
import jax
import jax.numpy as jnp
from jax.experimental import pallas as pl


def kernel(src_vectors, neighbor_vectors, W):
    raise NotImplementedError("write your pallas kernel here")


# speedup vs baseline: 42.9403x; 42.9403x over previous
import functools

import jax
import jax.numpy as jnp
from jax.experimental import pallas as pl


def _agg_kernel(src_ref, nbr_ref, w_ref, out_ref, *, k):
    s = src_ref.shape[0]
    d = src_ref.shape[1]
    nbr = nbr_ref[...]
    nsum = jnp.reshape(nbr, (s, k, d)).sum(axis=1)
    mean = (nsum + src_ref[...]) * (1.0 / (k + 1))
    out_ref[...] = jax.nn.relu(
        jnp.dot(mean, w_ref[...], preferred_element_type=jnp.float32)
    )


def kernel(src_vectors, neighbor_vectors, W):
    n_src, d = src_vectors.shape
    n_nbr = neighbor_vectors.shape[0]
    out_dim = W.shape[1]
    k = n_nbr // n_src

    S = 400
    grid = (n_src // S,)

    return pl.pallas_call(
        functools.partial(_agg_kernel, k=k),
        grid=grid,
        in_specs=[
            pl.BlockSpec((S, d), lambda i: (i, 0)),
            pl.BlockSpec((S * k, d), lambda i: (i, 0)),
            pl.BlockSpec((d, out_dim), lambda i: (0, 0)),
        ],
        out_specs=pl.BlockSpec((S, out_dim), lambda i: (i, 0)),
        out_shape=jax.ShapeDtypeStruct((n_src, out_dim), jnp.float32),
    )(src_vectors, neighbor_vectors, W)
